# baseline (speedup 1.0000x reference)
import jax
import jax.numpy as jnp
from jax import lax
from jax.experimental import pallas as pl
from jax.experimental.pallas import tpu as pltpu
from jax.experimental.pallas import tpu_sc as plsc

EPS_FACTOR = 2.0 * (1.0 + 0.1)

N_NODES = 10000
N_EDGES = 160000
D = 256
H = D // 2

NC = 2
NS = 16
CHUNK = 80
EDGES_PER_TILE = N_EDGES // NS
N_CHUNKS = EDGES_PER_TILE // CHUNK
WB = 80
NWB = N_NODES // WB
WB_PER_TILE = (NWB + NS - 1) // NS


def _sc_segment_sum_body(
    x20, src, dst, out, sbuf, db_0, db_1, db_2, rows_0, rows_1, rows_2, acc,
    g0, g1, g2, s0, s1, s2, d0, d1, d2,
):
  rows = [rows_0, rows_1, rows_2]
  db = [db_0, db_1, db_2]
  gsem = [g0, g1, g2]
  ssem = [s0, s1, s2]
  dsem = [d0, d1, d2]
  cid = lax.axis_index("c")
  sid = lax.axis_index("s")
  half_bits = cid * 8

  pltpu.sync_copy(src.at[pl.ds(sid * EDGES_PER_TILE, EDGES_PER_TILE)], sbuf)

  def remap_row(r, _):
    for j in range(CHUNK // 16):
      v = sbuf[pl.ds(r * CHUNK + j * 16, 16)]
      sbuf[pl.ds(r * CHUNK + j * 16, 16)] = ((v >> 3) << 4) | (v & 7) | half_bits
    return 0

  lax.fori_loop(0, N_CHUNKS, remap_row, 0)

  zero = jnp.zeros((16,), jnp.float32)

  def zero_row(r, _):
    for j in range(H // 16):
      rows_0[r, pl.ds(j * 16, 16)] = zero
    return 0

  lax.fori_loop(0, WB, zero_row, 0)
  for k in range(WB_PER_TILE):
    c = sid + NS * k

    @pl.when(c < NWB)
    def _():
      pltpu.sync_copy(rows_0, acc.at[pl.ds(c * WB, WB)])

  plsc.subcore_barrier()

  ebase = sid * EDGES_PER_TILE

  def gather(i, r):
    pltpu.async_copy(x20.at[sbuf.at[pl.ds(i * CHUNK, CHUNK)]], rows[r], gsem[r])

  def drain_gather(r):
    pltpu.make_async_copy(
        x20.at[sbuf.at[pl.ds(0, CHUNK)]], rows[r], gsem[r]
    ).wait()

  def scatter(r):
    pltpu.async_copy(rows[r], acc.at[db[r]], ssem[r], add=True)

  def wait_scatter(r):
    pltpu.make_async_copy(rows[r], acc.at[db[r]], ssem[r]).wait()

  def load_dst(i, r):
    pltpu.async_copy(dst.at[pl.ds(ebase + i * CHUNK, CHUNK)], db[r], dsem[r])

  def wait_dst(r):
    pltpu.make_async_copy(dst.at[pl.ds(0, CHUNK)], db[r], dsem[r]).wait()

  load_dst(0, 0)
  load_dst(1, 1)
  gather(0, 0)
  gather(1, 1)

  def step(k, _):
    j0 = 3 * k
    for r in range(3):
      j = j0 + r
      drain_gather(r)
      wait_dst(r)
      scatter(r)
      nr = (r + 2) % 3

      @pl.when(j >= 1)
      def _():
        wait_scatter(nr)

      load_dst(j + 2, nr)
      gather(j + 2, nr)
    return 0

  lax.fori_loop(0, (N_CHUNKS - 2) // 3, step, 0)
  for j in (N_CHUNKS - 2, N_CHUNKS - 1):
    r = j % 3
    drain_gather(r)
    wait_dst(r)
    scatter(r)
  for r in range(3):
    wait_scatter(r)
  plsc.subcore_barrier()

  for k in range(WB_PER_TILE):
    c = sid + NS * k

    @pl.when(c < NWB)
    def _():
      pltpu.sync_copy(acc.at[pl.ds(c * WB, WB)], rows_0)
      pltpu.sync_copy(rows_0, out.at[cid, pl.ds(c * WB, WB)])


@jax.jit
def _sc_segment_sum(x20, src, dst):
  mesh = plsc.VectorSubcoreMesh(
      core_axis_name="c", subcore_axis_name="s", num_cores=NC, num_subcores=NS
  )
  return pl.kernel(
      _sc_segment_sum_body,
      out_type=jax.ShapeDtypeStruct((NC, N_NODES, H), jnp.float32),
      mesh=mesh,
      scratch_types=(
          [pltpu.VMEM((EDGES_PER_TILE,), jnp.int32)]
          + [pltpu.VMEM((CHUNK,), jnp.int32)] * 3
          + [pltpu.VMEM((CHUNK, H), jnp.float32)] * 3
          + [pltpu.VMEM_SHARED((N_NODES, H), jnp.float32)]
          + [pltpu.SemaphoreType.DMA] * 9
      ),
  )(x20, src, dst)


def _dense_body(neigh_ref, x_ref, w_ref, b_ref, o_ref):
  lo = neigh_ref[0] + EPS_FACTOR * x_ref[:, :H]
  hi = neigh_ref[1] + EPS_FACTOR * x_ref[:, H:]
  dn = (((1,), (1,)), ((), ()))
  acc = lax.dot_general(
      lo, w_ref[:, :H], dn, preferred_element_type=jnp.float32
  )
  acc = acc + lax.dot_general(
      hi, w_ref[:, H:], dn, preferred_element_type=jnp.float32
  )
  o_ref[...] = jnp.maximum(acc + b_ref[...], 0.0)


@jax.jit
def _dense(neigh2, x, w, b2):
  bn = 504
  grid = (N_NODES // bn,)
  return pl.pallas_call(
      _dense_body,
      grid=grid,
      in_specs=[
          pl.BlockSpec((NC, bn, H), lambda i: (0, i, 0)),
          pl.BlockSpec((bn, D), lambda i: (i, 0)),
          pl.BlockSpec((D, D), lambda i: (0, 0)),
          pl.BlockSpec((1, D), lambda i: (0, 0)),
      ],
      out_specs=pl.BlockSpec((bn, D), lambda i: (i, 0)),
      out_shape=jax.ShapeDtypeStruct((N_NODES, D), jnp.float32),
  )(neigh2, x, w, b2)


def kernel(x, edge_index, W, b):
  e32 = edge_index.astype(jnp.int32)
  x20 = x.reshape(N_NODES // 8, 8, NC, H).transpose(0, 2, 1, 3).reshape(
      N_NODES * 2, H
  )
  neigh2 = _sc_segment_sum(x20, e32[0], e32[1])
  return _dense(neigh2, x, W, b.reshape(1, D))


# speedup vs baseline: 1.0191x; 1.0191x over previous
import jax
import jax.numpy as jnp
from jax import lax
from jax.experimental import pallas as pl
from jax.experimental.pallas import tpu as pltpu
from jax.experimental.pallas import tpu_sc as plsc

EPS_FACTOR = 2.0 * (1.0 + 0.1)

N_NODES = 10000
N_EDGES = 160000
D = 256
H = D // 2

NC = 2
NS = 16
CHUNK = 80
EDGES_PER_TILE = N_EDGES // NS
N_CHUNKS = EDGES_PER_TILE // CHUNK
WB = 80
NWB = N_NODES // WB
WB_PER_TILE = (NWB + NS - 1) // NS


def _sc_segment_sum_body(
    x20, src, dst, out, sbuf, db_0, db_1, db_2, rows_0, rows_1, rows_2, acc,
    g0, g1, g2, s0, s1, s2, d0, d1, d2,
):
  rows = [rows_0, rows_1, rows_2]
  db = [db_0, db_1, db_2]
  gsem = [g0, g1, g2]
  ssem = [s0, s1, s2]
  dsem = [d0, d1, d2]
  cid = lax.axis_index("c")
  sid = lax.axis_index("s")
  half_bits = cid * 8

  pltpu.sync_copy(src.at[pl.ds(sid * EDGES_PER_TILE, EDGES_PER_TILE)], sbuf)

  def remap_row(r, _):
    for j in range(CHUNK // 16):
      v = sbuf[pl.ds(r * CHUNK + j * 16, 16)]
      sbuf[pl.ds(r * CHUNK + j * 16, 16)] = ((v >> 3) << 4) | (v & 7) | half_bits
    return 0

  lax.fori_loop(0, N_CHUNKS, remap_row, 0)

  zero = jnp.zeros((16,), jnp.float32)

  def zero_row(r, _):
    for j in range(H // 16):
      rows_0[r, pl.ds(j * 16, 16)] = zero
    return 0

  lax.fori_loop(0, WB, zero_row, 0)
  for k in range(WB_PER_TILE):
    c = sid + NS * k

    @pl.when(c < NWB)
    def _():
      pltpu.sync_copy(rows_0, acc.at[pl.ds(c * WB, WB)])

  plsc.subcore_barrier()

  ebase = sid * EDGES_PER_TILE

  def gather(i, r):
    pltpu.async_copy(x20.at[sbuf.at[pl.ds(i * CHUNK, CHUNK)]], rows[r], gsem[r])

  def drain_gather(r):
    pltpu.make_async_copy(
        x20.at[sbuf.at[pl.ds(0, CHUNK)]], rows[r], gsem[r]
    ).wait()

  def scatter(r):
    pltpu.async_copy(rows[r], acc.at[db[r]], ssem[r], add=True)

  def wait_scatter(r):
    pltpu.make_async_copy(rows[r], acc.at[db[r]], ssem[r]).wait()

  def load_dst(i, r):
    pltpu.async_copy(dst.at[pl.ds(ebase + i * CHUNK, CHUNK)], db[r], dsem[r])

  def wait_dst(r):
    pltpu.make_async_copy(dst.at[pl.ds(0, CHUNK)], db[r], dsem[r]).wait()

  load_dst(0, 0)
  load_dst(1, 1)
  gather(0, 0)
  gather(1, 1)

  def step(k, _):
    j0 = 3 * k
    for r in range(3):
      j = j0 + r
      drain_gather(r)
      wait_dst(r)
      scatter(r)
      nr = (r + 2) % 3

      @pl.when(j >= 1)
      def _():
        wait_scatter(nr)

      load_dst(j + 2, nr)
      gather(j + 2, nr)
    return 0

  lax.fori_loop(0, (N_CHUNKS - 2) // 3, step, 0)
  for j in (N_CHUNKS - 2, N_CHUNKS - 1):
    r = j % 3
    drain_gather(r)
    wait_dst(r)
    scatter(r)
  for r in range(3):
    wait_scatter(r)
  plsc.subcore_barrier()

  for k in range(WB_PER_TILE):
    c = sid + NS * k

    @pl.when(c < NWB)
    def _():
      pltpu.sync_copy(acc.at[pl.ds(c * WB, WB)], rows_0)
      pltpu.sync_copy(rows_0, out.at[cid, pl.ds(c * WB, WB)])


@jax.jit
def _sc_segment_sum(x20, src, dst):
  mesh = plsc.VectorSubcoreMesh(
      core_axis_name="c", subcore_axis_name="s", num_cores=NC, num_subcores=NS
  )
  return pl.kernel(
      _sc_segment_sum_body,
      out_type=jax.ShapeDtypeStruct((NC, N_NODES, H), jnp.float32),
      mesh=mesh,
      scratch_types=(
          [pltpu.VMEM((EDGES_PER_TILE,), jnp.int32)]
          + [pltpu.VMEM((CHUNK,), jnp.int32)] * 3
          + [pltpu.VMEM((CHUNK, H), jnp.float32)] * 3
          + [pltpu.VMEM_SHARED((N_NODES, H), jnp.float32)]
          + [pltpu.SemaphoreType.DMA] * 9
      ),
  )(x20, src, dst)


def _dense_body(neigh_ref, x_ref, w_ref, b_ref, o_ref):
  lo = neigh_ref[0] + EPS_FACTOR * x_ref[:, :H]
  hi = neigh_ref[1] + EPS_FACTOR * x_ref[:, H:]
  dn = (((1,), (1,)), ((), ()))
  acc = lax.dot_general(
      lo, w_ref[:, :H], dn, preferred_element_type=jnp.float32
  )
  acc = acc + lax.dot_general(
      hi, w_ref[:, H:], dn, preferred_element_type=jnp.float32
  )
  o_ref[...] = jnp.maximum(acc + b_ref[...], 0.0)


@jax.jit
def _dense(neigh2, x, w, b2):
  bn = 5000
  grid = (N_NODES // bn,)
  return pl.pallas_call(
      _dense_body,
      grid=grid,
      in_specs=[
          pl.BlockSpec((NC, bn, H), lambda i: (0, i, 0)),
          pl.BlockSpec((bn, D), lambda i: (i, 0)),
          pl.BlockSpec((D, D), lambda i: (0, 0)),
          pl.BlockSpec((1, D), lambda i: (0, 0)),
      ],
      out_specs=pl.BlockSpec((bn, D), lambda i: (i, 0)),
      out_shape=jax.ShapeDtypeStruct((N_NODES, D), jnp.float32),
  )(neigh2, x, w, b2)


def kernel(x, edge_index, W, b):
  e32 = edge_index.astype(jnp.int32)
  x20 = x.reshape(N_NODES // 8, 8, NC, H).transpose(0, 2, 1, 3).reshape(
      N_NODES * 2, H
  )
  neigh2 = _sc_segment_sum(x20, e32[0], e32[1])
  return _dense(neigh2, x, W, b.reshape(1, D))


# speedup vs baseline: 1.0655x; 1.0456x over previous
import jax
import jax.numpy as jnp
from jax import lax
from jax.experimental import pallas as pl
from jax.experimental.pallas import tpu as pltpu
from jax.experimental.pallas import tpu_sc as plsc

EPS_FACTOR = 2.0 * (1.0 + 0.1)

N_NODES = 10000
N_EDGES = 160000
D = 256
H = D // 2

NC = 2
NS = 16
CHUNK = 80
EDGES_PER_TILE = N_EDGES // NS
N_CHUNKS = EDGES_PER_TILE // CHUNK
WB = 80
NWB = N_NODES // WB
WB_PER_TILE = (NWB + NS - 1) // NS


def _sc_segment_sum_body(
    x20, src, dst, out, sbuf, db_0, db_1, db_2, rows_0, rows_1, rows_2, acc,
    g0, g1, g2, s0, s1, s2, d0, d1, d2,
):
  rows = [rows_0, rows_1, rows_2]
  db = [db_0, db_1, db_2]
  gsem = [g0, g1, g2]
  ssem = [s0, s1, s2]
  dsem = [d0, d1, d2]
  cid = lax.axis_index("c")
  sid = lax.axis_index("s")
  half_bits = cid * 8

  pltpu.sync_copy(src.at[pl.ds(sid * EDGES_PER_TILE, EDGES_PER_TILE)], sbuf)

  def remap_row(r, _):
    for j in range(CHUNK // 16):
      v = sbuf[pl.ds(r * CHUNK + j * 16, 16)]
      sbuf[pl.ds(r * CHUNK + j * 16, 16)] = ((v >> 3) << 4) | (v & 7) | half_bits
    return 0

  lax.fori_loop(0, N_CHUNKS, remap_row, 0)

  zero = jnp.zeros((16,), jnp.float32)

  def zero_row(r, _):
    for j in range(H // 16):
      rows_0[r, pl.ds(j * 16, 16)] = zero
    return 0

  lax.fori_loop(0, WB, zero_row, 0)
  nz = 0
  for k in range(WB_PER_TILE):
    c = sid + NS * k

    @pl.when(c < NWB)
    def _():
      pltpu.async_copy(rows_0, acc.at[pl.ds(c * WB, WB)], g0)

    nz += 1
  for k in range(nz):
    @pl.when(sid + NS * k < NWB)
    def _():
      pltpu.make_async_copy(rows_0, acc.at[pl.ds(0, WB)], g0).wait()

  plsc.subcore_barrier()

  ebase = sid * EDGES_PER_TILE

  def gather(i, r):
    pltpu.async_copy(x20.at[sbuf.at[pl.ds(i * CHUNK, CHUNK)]], rows[r], gsem[r])

  def drain_gather(r):
    pltpu.make_async_copy(
        x20.at[sbuf.at[pl.ds(0, CHUNK)]], rows[r], gsem[r]
    ).wait()

  def scatter(r):
    pltpu.async_copy(rows[r], acc.at[db[r]], ssem[r], add=True)

  def wait_scatter(r):
    pltpu.make_async_copy(rows[r], acc.at[db[r]], ssem[r]).wait()

  def load_dst(i, r):
    pltpu.async_copy(dst.at[pl.ds(ebase + i * CHUNK, CHUNK)], db[r], dsem[r])

  def wait_dst(r):
    pltpu.make_async_copy(dst.at[pl.ds(0, CHUNK)], db[r], dsem[r]).wait()

  load_dst(0, 0)
  load_dst(1, 1)
  gather(0, 0)
  gather(1, 1)

  def step(k, _):
    j0 = 3 * k
    for r in range(3):
      j = j0 + r
      drain_gather(r)
      wait_dst(r)
      scatter(r)
      nr = (r + 2) % 3

      @pl.when(j >= 1)
      def _():
        wait_scatter(nr)

      load_dst(j + 2, nr)
      gather(j + 2, nr)
    return 0

  lax.fori_loop(0, (N_CHUNKS - 2) // 3, step, 0)
  for j in (N_CHUNKS - 2, N_CHUNKS - 1):
    r = j % 3
    drain_gather(r)
    wait_dst(r)
    scatter(r)
  for r in range(3):
    wait_scatter(r)
  plsc.subcore_barrier()

  def rd(k, r):
    c = sid + NS * k

    @pl.when(c < NWB)
    def _():
      pltpu.async_copy(acc.at[pl.ds(c * WB, WB)], rows[r], gsem[r])

  def wr(k, r):
    c = sid + NS * k

    @pl.when(c < NWB)
    def _():
      pltpu.make_async_copy(acc.at[pl.ds(0, WB)], rows[r], gsem[r]).wait()
      pltpu.async_copy(rows[r], out.at[cid, pl.ds(c * WB, WB)], ssem[r])

  def wrw(k, r):
    c = sid + NS * k

    @pl.when(c < NWB)
    def _():
      pltpu.make_async_copy(
          rows[r], out.at[cid, pl.ds(0, WB)], ssem[r]
      ).wait()

  rd(0, 0)
  for k in range(WB_PER_TILE):
    r = k % 2
    if k >= 1:
      wrw(k - 1, 1 - r)
    if k + 1 < WB_PER_TILE:
      rd(k + 1, 1 - r)
    wr(k, r)
  wrw(WB_PER_TILE - 1, (WB_PER_TILE - 1) % 2)


@jax.jit
def _sc_segment_sum(x20, src, dst):
  mesh = plsc.VectorSubcoreMesh(
      core_axis_name="c", subcore_axis_name="s", num_cores=NC, num_subcores=NS
  )
  return pl.kernel(
      _sc_segment_sum_body,
      out_type=jax.ShapeDtypeStruct((NC, N_NODES, H), jnp.float32),
      mesh=mesh,
      scratch_types=(
          [pltpu.VMEM((EDGES_PER_TILE,), jnp.int32)]
          + [pltpu.VMEM((CHUNK,), jnp.int32)] * 3
          + [pltpu.VMEM((CHUNK, H), jnp.float32)] * 3
          + [pltpu.VMEM_SHARED((N_NODES, H), jnp.float32)]
          + [pltpu.SemaphoreType.DMA] * 9
      ),
  )(x20, src, dst)


def _dense_body(neigh_ref, x_ref, w_ref, b_ref, o_ref):
  lo = neigh_ref[0] + EPS_FACTOR * x_ref[:, :H]
  hi = neigh_ref[1] + EPS_FACTOR * x_ref[:, H:]
  dn = (((1,), (1,)), ((), ()))
  acc = lax.dot_general(
      lo, w_ref[:, :H], dn, preferred_element_type=jnp.float32
  )
  acc = acc + lax.dot_general(
      hi, w_ref[:, H:], dn, preferred_element_type=jnp.float32
  )
  o_ref[...] = jnp.maximum(acc + b_ref[...], 0.0)


@jax.jit
def _dense(neigh2, x, w, b2):
  bn = 2000
  grid = (N_NODES // bn,)
  return pl.pallas_call(
      _dense_body,
      grid=grid,
      in_specs=[
          pl.BlockSpec((NC, bn, H), lambda i: (0, i, 0)),
          pl.BlockSpec((bn, D), lambda i: (i, 0)),
          pl.BlockSpec((D, D), lambda i: (0, 0)),
          pl.BlockSpec((1, D), lambda i: (0, 0)),
      ],
      out_specs=pl.BlockSpec((bn, D), lambda i: (i, 0)),
      out_shape=jax.ShapeDtypeStruct((N_NODES, D), jnp.float32),
  )(neigh2, x, w, b2)


def kernel(x, edge_index, W, b):
  e32 = edge_index.astype(jnp.int32)
  x20 = x.reshape(N_NODES // 8, 8, NC, H).transpose(0, 2, 1, 3).reshape(
      N_NODES * 2, H
  )
  neigh2 = _sc_segment_sum(x20, e32[0], e32[1])
  return _dense(neigh2, x, W, b.reshape(1, D))
